# baseline (speedup 1.0000x reference)
import functools

import jax
import jax.numpy as jnp
from jax.experimental import pallas as pl
from jax.experimental.pallas import tpu as pltpu

SMOOTH = 8.0
VOCAB = 50000
Q = 512
N = 65536
D = 512

N_BLK = 2048


def _kern_body(h_ref, c_ref, o_ref):
    h = h_ref[...]
    c = c_ref[...]
    hf = h.astype(jnp.float32)
    cf = c.astype(jnp.float32)
    qsq = jnp.sum(hf * hf, axis=1, keepdims=True)
    ksq = jnp.sum(cf * cf, axis=1)[None, :]
    dots = jax.lax.dot_general(
        h, c, dimension_numbers=(((1,), (1,)), ((), ())),
        preferred_element_type=jnp.float32)
    sq = jnp.maximum(qsq + ksq - 2.0 * dots, 0.0)
    o_ref[...] = jnp.exp(jnp.sqrt(sq) * (1.0 / SMOOTH))


def _kern_matrix(h_bf, c_bf):
    return pl.pallas_call(
        _kern_body,
        grid=(N // N_BLK,),
        in_specs=[
            pl.BlockSpec((Q, D), lambda i: (0, 0)),
            pl.BlockSpec((N_BLK, D), lambda i: (i, 0)),
        ],
        out_specs=pl.BlockSpec((Q, N_BLK), lambda i: (0, i)),
        out_shape=jax.ShapeDtypeStruct((Q, N), jnp.float32),
    )(h_bf, c_bf)


def kernel(h_t, cache_hiddens, cache_words):
    h_bf = h_t.astype(jnp.bfloat16)
    c_bf = cache_hiddens.astype(jnp.bfloat16)
    kern = _kern_matrix(h_bf, c_bf)
    cache_p = jnp.zeros((Q, VOCAB), jnp.float32).at[:, cache_words].add(kern)
    return jax.nn.log_softmax(cache_p, axis=-1)
